# baseline (speedup 1.0000x reference)
import jax
import jax.numpy as jnp
from jax.experimental import pallas as pl
from jax.experimental.pallas import tpu as pltpu

_K = 4


def _make_body(nb, A, attrs, H, W):
    def body(x_hbm, o_hbm, inb, outb, insem, insem2, outsem):
        i = pl.program_id(0)

        half = attrs // 2

        def in_copy_a(j):
            return pltpu.make_async_copy(
                x_hbm.at[j // A, pl.ds((j % A) * attrs, half)],
                inb.at[j % _K, pl.ds(0, half)],
                insem.at[j % _K],
            )

        def in_copy_b(j):
            return pltpu.make_async_copy(
                x_hbm.at[j // A, pl.ds((j % A) * attrs + half, attrs - half)],
                inb.at[j % _K, pl.ds(half, attrs - half)],
                insem2.at[j % _K],
            )

        def start_in(j):
            in_copy_a(j).start()
            in_copy_b(j).start()

        def wait_in(j):
            in_copy_a(j).wait()
            in_copy_b(j).wait()

        def out_copy(j):
            return pltpu.make_async_copy(
                outb.at[j % _K],
                o_hbm.at[j // A, j % A],
                outsem.at[j % _K],
            )

        @pl.when(i == 0)
        def _():
            for j in range(_K):
                start_in(j)

        @pl.when((i > 0) & (i + _K - 1 < nb))
        def _():
            start_in(i + _K - 1)

        wait_in(i)

        @pl.when(i >= _K)
        def _():
            out_copy(i - _K).wait()

        outb[i % _K] = jnp.transpose(inb[i % _K], (1, 2, 0))

        out_copy(i).start()

        @pl.when(i == nb - 1)
        def _():
            for d in range(_K):
                out_copy(i - _K + 1 + d).wait()

    return body


def kernel(input):
    bs, C, H, W = input.shape
    A = 3
    attrs = C // A
    nb = bs * A

    return pl.pallas_call(
        _make_body(nb, A, attrs, H, W),
        grid=(nb,),
        in_specs=[pl.BlockSpec(memory_space=pl.ANY)],
        out_specs=pl.BlockSpec(memory_space=pl.ANY),
        out_shape=jax.ShapeDtypeStruct((bs, A, H, W, attrs), input.dtype),
        scratch_shapes=[
            pltpu.VMEM((_K, attrs, H, W), input.dtype),
            pltpu.VMEM((_K, H, W, attrs), input.dtype),
            pltpu.SemaphoreType.DMA((_K,)),
            pltpu.SemaphoreType.DMA((_K,)),
            pltpu.SemaphoreType.DMA((_K,)),
        ],
        compiler_params=pltpu.CompilerParams(
            dimension_semantics=("arbitrary",),
        ),
    )(input)


# speedup vs baseline: 1.0012x; 1.0012x over previous
import jax
import jax.numpy as jnp
from jax.experimental import pallas as pl
from jax.experimental.pallas import tpu as pltpu

_K = 4


def _make_body(nb, A, attrs, H, W):
    def body(x_hbm, o_hbm, inb, outb, insem, outsem):
        i = pl.program_id(0)

        def in_copy(j):
            return pltpu.make_async_copy(
                x_hbm.at[j // A, pl.ds((j % A) * attrs, attrs)],
                inb.at[j % _K],
                insem.at[j % _K],
            )

        def out_copy(j):
            return pltpu.make_async_copy(
                outb.at[j % _K],
                o_hbm.at[j // A, j % A],
                outsem.at[j % _K],
            )

        @pl.when(i == 0)
        def _():
            for j in range(_K):
                in_copy(j).start()

        @pl.when((i > 0) & (i + _K - 1 < nb))
        def _():
            in_copy(i + _K - 1).start()

        in_copy(i).wait()

        @pl.when(i >= _K)
        def _():
            out_copy(i - _K).wait()

        outb[i % _K] = jnp.transpose(inb[i % _K], (1, 2, 0))

        out_copy(i).start()

        @pl.when(i == nb - 1)
        def _():
            for d in range(_K):
                out_copy(i - _K + 1 + d).wait()

    return body


def kernel(input):
    bs, C, H, W = input.shape
    A = 3
    attrs = C // A
    nb = bs * A

    return pl.pallas_call(
        _make_body(nb, A, attrs, H, W),
        grid=(nb,),
        in_specs=[pl.BlockSpec(memory_space=pl.ANY)],
        out_specs=pl.BlockSpec(memory_space=pl.ANY),
        out_shape=jax.ShapeDtypeStruct((bs, A, H, W, attrs), input.dtype),
        scratch_shapes=[
            pltpu.VMEM((_K, attrs, H, W), input.dtype),
            pltpu.VMEM((_K, H, W, attrs), input.dtype),
            pltpu.SemaphoreType.DMA((_K,)),
            pltpu.SemaphoreType.DMA((_K,)),
        ],
        compiler_params=pltpu.CompilerParams(
            dimension_semantics=("arbitrary",),
        ),
    )(input)
